# Initial kernel scaffold; baseline (speedup 1.0000x reference)
import jax
import jax.numpy as jnp
from jax.experimental import pallas as pl


def kernel(x, embeddings):
    raise NotImplementedError("write your pallas kernel here")


# speedup vs baseline: 2.2104x; 2.2104x over previous
import jax
import jax.numpy as jnp
from jax.experimental import pallas as pl

_NUM_EMB = 1024
_DIM = 64
_BLK = 512


def _vq_body(x_ref, emb_ref, out_ref):
    xb = x_ref[:]
    emb = emb_ref[:]
    sim = jax.lax.dot_general(
        xb, emb, (((1,), (0,)), ((), ())), preferred_element_type=jnp.float32
    )
    e2 = jnp.sum(emb * emb, axis=0, keepdims=True)
    dist = e2 - 2.0 * sim
    minval = jnp.min(dist, axis=1, keepdims=True)
    lanes = jax.lax.broadcasted_iota(jnp.int32, dist.shape, 1)
    idx = jnp.min(jnp.where(dist == minval, lanes, _NUM_EMB), axis=1, keepdims=True)
    onehot = (lanes == idx).astype(jnp.float32)
    out_ref[:] = jax.lax.dot_general(
        onehot, emb, (((1,), (1,)), ((), ())), preferred_element_type=jnp.float32
    )


def kernel(x, embeddings):
    flat = x.reshape(-1, _DIM)
    n = flat.shape[0]
    grid = (n // _BLK,)
    out = pl.pallas_call(
        _vq_body,
        grid=grid,
        in_specs=[
            pl.BlockSpec((_BLK, _DIM), lambda i: (i, 0)),
            pl.BlockSpec((_DIM, _NUM_EMB), lambda i: (0, 0)),
        ],
        out_specs=pl.BlockSpec((_BLK, _DIM), lambda i: (i, 0)),
        out_shape=jax.ShapeDtypeStruct((n, _DIM), jnp.float32),
    )(flat, embeddings)
    return out.reshape(x.shape)
